# baseline (speedup 1.0000x reference)
import functools

import jax
import jax.numpy as jnp
from jax import lax
from jax.experimental import pallas as pl
from jax.experimental.pallas import tpu as pltpu
from jax.experimental.pallas import tpu_sc as plsc

_VOCAB = 1000000
_EMBED = 64
_BATCH = 4096
_SEQ = 200

_NC = 2
_NS = 16
_NW = _NC * _NS
_RPW = _BATCH // _NW
_C0 = 128
_C1 = _SEQ - _C0
_NBUF = 4
_DEPTH = 2


@functools.partial(
    pl.kernel,
    mesh=plsc.VectorSubcoreMesh(core_axis_name="c", subcore_axis_name="s"),
    out_type=jax.ShapeDtypeStruct((_BATCH, _SEQ, _EMBED), jnp.float32),
    scratch_types=[
        pltpu.VMEM((_RPW, _SEQ), jnp.int32),
        pltpu.VMEM((_NBUF, _SEQ, _EMBED), jnp.float32),
        pltpu.SemaphoreType.DMA,
        pltpu.SemaphoreType.DMA,
    ],
    compiler_params=pltpu.CompilerParams(use_tc_tiling_on_sc=False),
)
def _gather_kernel(idx_hbm, table_hbm, out_hbm, idx_v, rows_v, sem_g, sem_s):
    wid = lax.axis_index("s") * _NC + lax.axis_index("c")
    b0 = wid * _RPW

    pltpu.sync_copy(idx_hbm.at[pl.ds(b0, _RPW)], idx_v)

    def gathers_start(r, slot):
        pltpu.async_copy(
            table_hbm.at[idx_v.at[r, pl.ds(0, _C0)]],
            rows_v.at[slot, pl.ds(0, _C0)],
            sem_g,
        )
        pltpu.async_copy(
            table_hbm.at[idx_v.at[r, pl.ds(_C0, _C1)]],
            rows_v.at[slot, pl.ds(_C0, _C1)],
            sem_g,
        )

    def gathers_wait(slot):
        pltpu.make_async_copy(
            table_hbm.at[idx_v.at[0, pl.ds(0, _C0)]],
            rows_v.at[slot, pl.ds(0, _C0)],
            sem_g,
        ).wait()
        pltpu.make_async_copy(
            table_hbm.at[idx_v.at[0, pl.ds(_C0, _C1)]],
            rows_v.at[slot, pl.ds(_C0, _C1)],
            sem_g,
        ).wait()

    def store_start(r, slot):
        pltpu.async_copy(rows_v.at[slot], out_hbm.at[b0 + r], sem_s)

    def store_wait(slot):
        pltpu.make_async_copy(rows_v.at[slot], out_hbm.at[0], sem_s).wait()

    for r in range(_DEPTH):
        gathers_start(r, r)

    def body(r, carry):
        slot = r % _NBUF
        gathers_wait(slot)
        store_start(r, slot)

        r2 = r + _DEPTH

        @pl.when(r2 < _RPW)
        def _():
            slot2 = (r + _DEPTH) % _NBUF

            @pl.when(r2 >= _NBUF)
            def _():
                store_wait(slot2)

            gathers_start(r2, slot2)

        return carry

    lax.fori_loop(0, _RPW, body, 0)

    for _ in range(_NBUF):
        store_wait(0)


def kernel(word_vector, table):
    return _gather_kernel(word_vector.astype(jnp.int32), table)
